# baseline (speedup 1.0000x reference)
import functools

import jax
import jax.numpy as jnp
from jax import lax
from jax.experimental import pallas as pl
from jax.experimental.pallas import tpu as pltpu
from jax.experimental.pallas import tpu_sc as plsc

N_ROWS = 100000
N_FEAT = 128
N_SEG = 64
BLK = 80
N_BLOCKS = N_ROWS // BLK
SUB = 5
SB_ROWS = BLK * SUB
N_SB = N_ROWS // SB_ROWS
N_CORES = 2
N_SUBCORES = 16
N_WORKERS = N_CORES * N_SUBCORES
SB_PER_W = -(-N_SB // N_WORKERS)
LANES = 16
FCHUNKS = N_FEAT // LANES
ROW_UNROLL = 4


def _sc_segment_sum(x, batch32):
    mesh = plsc.VectorSubcoreMesh(core_axis_name="c", subcore_axis_name="s")

    @functools.partial(
        pl.kernel,
        mesh=mesh,
        out_type=[
            jax.ShapeDtypeStruct((N_CORES, N_SEG, N_FEAT), jnp.float32),
            jax.ShapeDtypeStruct((N_CORES, N_SEG), jnp.float32),
        ],
        scratch_types=[
            pltpu.VMEM((SB_ROWS, N_FEAT), jnp.float32),
            pltpu.VMEM((SB_ROWS, N_FEAT), jnp.float32),
        ] + [pltpu.VMEM((BLK,), jnp.int32)] * (2 * SUB) + [
            pltpu.VMEM((SB_ROWS,), jnp.int32),
            pltpu.VMEM((SB_ROWS,), jnp.int32),
            pltpu.VMEM((BLK,), jnp.float32),
            pltpu.VMEM((N_SEG, N_FEAT), jnp.float32),
            pltpu.VMEM((N_SEG,), jnp.int32),
            pltpu.VMEM((N_SEG, N_FEAT), jnp.float32),
            pltpu.VMEM((N_SEG,), jnp.float32),
            pltpu.VMEM_SHARED((N_SEG, N_FEAT), jnp.float32),
            pltpu.VMEM_SHARED((N_SEG,), jnp.float32),
            pltpu.SemaphoreType.DMA,
            pltpu.SemaphoreType.DMA,
            pltpu.SemaphoreType.DMA,
            pltpu.SemaphoreType.DMA,
        ],
    )
    def seg_sum(x_hbm, b_hbm, sums_hbm, cnts_hbm,
                xb0, xb1, *rest):
        ibv0 = rest[0:SUB]
        ibv1 = rest[SUB:2 * SUB]
        (iba0, iba1, ones, acc, iden, zrow, zcnt, acc_sh, cnt_sh,
         sl0, sl1, ss0, ss1) = rest[2 * SUB:]
        cid = lax.axis_index("c")
        sid = lax.axis_index("s")
        wid = sid * N_CORES + cid

        z16 = jnp.zeros((LANES,), dtype=jnp.float32)
        one16 = jnp.full((LANES,), 1.0, dtype=jnp.float32)
        iota16 = lax.iota(jnp.int32, LANES)

        for k in range(N_SEG // LANES):
            iden[pl.ds(k * LANES, LANES)] = iota16 + (k * LANES)
        for k in range(BLK // LANES):
            ones[pl.ds(k * LANES, LANES)] = one16

        def zero_acc_row(r, carry):
            for j in range(FCHUNKS):
                acc[r, pl.ds(j * LANES, LANES)] = z16
            return carry

        lax.fori_loop(0, N_SEG, zero_acc_row, 0)

        @pl.when(sid == 0)
        def _init():
            for k in range(N_SEG // LANES):
                zcnt[pl.ds(k * LANES, LANES)] = z16

            def zero_zrow(r, carry):
                for j in range(FCHUNKS):
                    zrow[r, pl.ds(j * LANES, LANES)] = z16
                return carry

            lax.fori_loop(0, N_SEG, zero_zrow, 0)
            pltpu.sync_copy(zrow, acc_sh)
            pltpu.sync_copy(zcnt, cnt_sh)

        plsc.subcore_barrier()

        def srcs(i):
            sb = wid + i * N_WORKERS
            off = sb * SB_ROWS
            return (x_hbm.at[pl.ds(off, SB_ROWS)],
                    [b_hbm.at[pl.ds(off + j * BLK, BLK)] for j in range(SUB)])

        def load_start(i, xb, ibv, iba, sl):
            sb = wid + i * N_WORKERS

            @pl.when(sb < N_SB)
            def _():
                xs, bs = srcs(i)
                pltpu.async_copy(xs, xb, sl)
                pltpu.async_copy(b_hbm.at[pl.ds(sb * SB_ROWS, SB_ROWS)],
                                 iba, sl)
                for j in range(SUB):
                    pltpu.async_copy(bs[j], ibv[j], sl)

        cnt_descs = {}

        def accumulate_sb(xb, iba):
            def grp_body(g, carry):
                idvec = iba[pl.ds(g * LANES, LANES)]
                for u in range(LANES):
                    b = idvec[u]
                    rg = g * LANES + u
                    for j in range(FCHUNKS):
                        v = xb[rg, pl.ds(j * LANES, LANES)]
                        plsc.addupdate(acc.at[b, pl.ds(j * LANES, LANES)], v)
                return carry

            lax.fori_loop(0, SB_ROWS // LANES, grp_body, 0)

        def load_wait_and_accum(i, xb, ibv, iba, sl, ss):
            sb = wid + i * N_WORKERS

            @pl.when(sb < N_SB)
            def _():
                xs, bs = srcs(i)
                pltpu.make_async_copy(xs, xb, sl).wait()
                pltpu.make_async_copy(b_hbm.at[pl.ds(sb * SB_ROWS, SB_ROWS)],
                                      iba, sl).wait()
                for j in range(SUB):
                    pltpu.make_async_copy(bs[j], ibv[j], sl).wait()
                ds = []
                for j in range(SUB):
                    ds.append(pltpu.async_copy(ones, cnt_sh.at[ibv[j]],
                                               ss, add=True))
                cnt_descs[i] = ds
                accumulate_sb(xb, iba)

        def cnt_drain(i):
            if i < 0 or i not in cnt_descs:
                return
            sb = wid + i * N_WORKERS

            @pl.when(sb < N_SB)
            def _():
                for d in cnt_descs[i]:
                    d.wait()

        bufs = [(xb0, ibv0, iba0, sl0, ss0), (xb1, ibv1, iba1, sl1, ss1)]
        load_start(0, *bufs[0][:4])
        for i in range(SB_PER_W):
            if i + 1 < SB_PER_W:
                cnt_drain(i - 1)
                load_start(i + 1, *bufs[(i + 1) % 2][:4])
            load_wait_and_accum(i, *bufs[i % 2])
        cnt_drain(SB_PER_W - 2)
        cnt_drain(SB_PER_W - 1)

        pltpu.sync_copy(acc, acc_sh.at[iden], add=True)

        plsc.subcore_barrier()

        @pl.when(sid == 0)
        def _emit():
            pltpu.sync_copy(acc_sh, sums_hbm.at[cid])
            pltpu.sync_copy(cnt_sh, cnts_hbm.at[cid])

    return seg_sum(x, batch32)


def _combine_kernel(sums_ref, cnts_ref, out_ref):
    s = sums_ref[0] + sums_ref[1]
    c = jnp.maximum(cnts_ref[0] + cnts_ref[1], 1.0)
    out_ref[...] = s / c[:, None]


def _tc_combine(sums, cnts):
    return pl.pallas_call(
        _combine_kernel,
        out_shape=jax.ShapeDtypeStruct((N_SEG, N_FEAT), jnp.float32),
    )(sums, cnts)


@jax.jit
def kernel(x, batch):
    batch32 = batch.astype(jnp.int32)
    sums, cnts = _sc_segment_sum(x, batch32)
    return _tc_combine(sums, cnts)


# speedup vs baseline: 1.6079x; 1.6079x over previous
import functools

import jax
import jax.numpy as jnp
from jax import lax
from jax.experimental import pallas as pl
from jax.experimental.pallas import tpu as pltpu
from jax.experimental.pallas import tpu_sc as plsc

N_ROWS = 100000
N_FEAT = 128
N_SEG = 64
BLK = 80
N_BLOCKS = N_ROWS // BLK
SUB = 5
SB_ROWS = BLK * SUB
N_SB = N_ROWS // SB_ROWS
N_CORES = 2
N_SUBCORES = 16
N_WORKERS = N_CORES * N_SUBCORES
SB_PER_W = -(-N_SB // N_WORKERS)
LANES = 16
FCHUNKS = N_FEAT // LANES
ROW_UNROLL = 4


def _sc_segment_sum(x, batch32):
    mesh = plsc.VectorSubcoreMesh(core_axis_name="c", subcore_axis_name="s")

    @functools.partial(
        pl.kernel,
        mesh=mesh,
        out_type=[
            jax.ShapeDtypeStruct((N_CORES, N_SEG, N_FEAT), jnp.float32),
            jax.ShapeDtypeStruct((N_CORES, N_SEG), jnp.float32),
        ],
        scratch_types=[
            pltpu.VMEM((SB_ROWS, N_FEAT), jnp.float32),
            pltpu.VMEM((SB_ROWS, N_FEAT), jnp.float32),
        ] + [pltpu.VMEM((BLK,), jnp.int32)] * (2 * SUB) + [
            pltpu.VMEM((SB_ROWS,), jnp.int32),
            pltpu.VMEM((SB_ROWS,), jnp.int32),
            pltpu.VMEM((BLK,), jnp.float32),
            pltpu.VMEM((N_SEG, N_FEAT), jnp.float32),
            pltpu.VMEM((N_SEG,), jnp.int32),
            pltpu.VMEM((N_SEG, N_FEAT), jnp.float32),
            pltpu.VMEM((N_SEG,), jnp.float32),
            pltpu.VMEM_SHARED((N_SEG, N_FEAT), jnp.float32),
            pltpu.VMEM_SHARED((N_SEG,), jnp.float32),
            pltpu.SemaphoreType.DMA,
            pltpu.SemaphoreType.DMA,
            pltpu.SemaphoreType.DMA,
            pltpu.SemaphoreType.DMA,
        ],
    )
    def seg_sum(x_hbm, b_hbm, sums_hbm, cnts_hbm,
                xb0, xb1, *rest):
        ibv0 = rest[0:SUB]
        ibv1 = rest[SUB:2 * SUB]
        (iba0, iba1, ones, acc, iden, zrow, zcnt, acc_sh, cnt_sh,
         sl0, sl1, ss0, ss1) = rest[2 * SUB:]
        cid = lax.axis_index("c")
        sid = lax.axis_index("s")
        wid = sid * N_CORES + cid

        z16 = jnp.zeros((LANES,), dtype=jnp.float32)
        one16 = jnp.full((LANES,), 1.0, dtype=jnp.float32)
        iota16 = lax.iota(jnp.int32, LANES)

        for k in range(N_SEG // LANES):
            iden[pl.ds(k * LANES, LANES)] = iota16 + (k * LANES)
        for k in range(BLK // LANES):
            ones[pl.ds(k * LANES, LANES)] = one16

        def zero_acc_row(r, carry):
            for j in range(FCHUNKS):
                acc[r, pl.ds(j * LANES, LANES)] = z16
            return carry

        lax.fori_loop(0, N_SEG, zero_acc_row, 0)

        @pl.when(sid == 0)
        def _init():
            for k in range(N_SEG // LANES):
                zcnt[pl.ds(k * LANES, LANES)] = z16

            def zero_zrow(r, carry):
                for j in range(FCHUNKS):
                    zrow[r, pl.ds(j * LANES, LANES)] = z16
                return carry

            lax.fori_loop(0, N_SEG, zero_zrow, 0)
            pltpu.sync_copy(zrow, acc_sh)
            pltpu.sync_copy(zcnt, cnt_sh)

        plsc.subcore_barrier()

        def srcs(i):
            sb = wid + i * N_WORKERS
            off = sb * SB_ROWS
            return (x_hbm.at[pl.ds(off, SB_ROWS)],
                    [b_hbm.at[pl.ds(off + j * BLK, BLK)] for j in range(SUB)])

        def load_start(i, xb, ibv, iba, sl):
            sb = wid + i * N_WORKERS

            @pl.when(sb < N_SB)
            def _():
                xs, bs = srcs(i)
                pltpu.async_copy(xs, xb, sl)
                pltpu.async_copy(b_hbm.at[pl.ds(sb * SB_ROWS, SB_ROWS)],
                                 iba, sl)
                for j in range(SUB):
                    pltpu.async_copy(bs[j], ibv[j], sl)

        cnt_descs = {}

        def accumulate_sb(xb, iba):
            def grp_body(g, carry):
                idvec = iba[pl.ds(g * LANES, LANES)]
                b0 = idvec[0]
                b15 = idvec[LANES - 1]

                @pl.when(b0 == b15)
                def _uniform():
                    for j in range(FCHUNKS):
                        s = xb[g * LANES, pl.ds(j * LANES, LANES)]
                        for u in range(1, LANES):
                            s = s + xb[g * LANES + u, pl.ds(j * LANES, LANES)]
                        plsc.addupdate(acc.at[b0, pl.ds(j * LANES, LANES)], s)

                @pl.when(b0 != b15)
                def _boundary():
                    for u in range(LANES):
                        b = idvec[u]
                        rg = g * LANES + u
                        for j in range(FCHUNKS):
                            v = xb[rg, pl.ds(j * LANES, LANES)]
                            plsc.addupdate(
                                acc.at[b, pl.ds(j * LANES, LANES)], v)
                return carry

            lax.fori_loop(0, SB_ROWS // LANES, grp_body, 0)

        def load_wait_and_accum(i, xb, ibv, iba, sl, ss):
            sb = wid + i * N_WORKERS

            @pl.when(sb < N_SB)
            def _():
                xs, bs = srcs(i)
                pltpu.make_async_copy(xs, xb, sl).wait()
                pltpu.make_async_copy(b_hbm.at[pl.ds(sb * SB_ROWS, SB_ROWS)],
                                      iba, sl).wait()
                for j in range(SUB):
                    pltpu.make_async_copy(bs[j], ibv[j], sl).wait()
                ds = []
                for j in range(SUB):
                    ds.append(pltpu.async_copy(ones, cnt_sh.at[ibv[j]],
                                               ss, add=True))
                cnt_descs[i] = ds
                accumulate_sb(xb, iba)

        def cnt_drain(i):
            if i < 0 or i not in cnt_descs:
                return
            sb = wid + i * N_WORKERS

            @pl.when(sb < N_SB)
            def _():
                for d in cnt_descs[i]:
                    d.wait()

        bufs = [(xb0, ibv0, iba0, sl0, ss0), (xb1, ibv1, iba1, sl1, ss1)]
        load_start(0, *bufs[0][:4])
        for i in range(SB_PER_W):
            if i + 1 < SB_PER_W:
                cnt_drain(i - 1)
                load_start(i + 1, *bufs[(i + 1) % 2][:4])
            load_wait_and_accum(i, *bufs[i % 2])
        cnt_drain(SB_PER_W - 2)
        cnt_drain(SB_PER_W - 1)

        pltpu.sync_copy(acc, acc_sh.at[iden], add=True)

        plsc.subcore_barrier()

        @pl.when(sid == 0)
        def _emit():
            pltpu.sync_copy(acc_sh, sums_hbm.at[cid])
            pltpu.sync_copy(cnt_sh, cnts_hbm.at[cid])

    return seg_sum(x, batch32)


def _combine_kernel(sums_ref, cnts_ref, out_ref):
    s = sums_ref[0] + sums_ref[1]
    c = jnp.maximum(cnts_ref[0] + cnts_ref[1], 1.0)
    out_ref[...] = s / c[:, None]


def _tc_combine(sums, cnts):
    return pl.pallas_call(
        _combine_kernel,
        out_shape=jax.ShapeDtypeStruct((N_SEG, N_FEAT), jnp.float32),
    )(sums, cnts)


@jax.jit
def kernel(x, batch):
    batch32 = batch.astype(jnp.int32)
    sums, cnts = _sc_segment_sum(x, batch32)
    return _tc_combine(sums, cnts)


# speedup vs baseline: 2.0242x; 1.2589x over previous
import functools

import jax
import jax.numpy as jnp
from jax import lax
from jax.experimental import pallas as pl
from jax.experimental.pallas import tpu as pltpu
from jax.experimental.pallas import tpu_sc as plsc

N_ROWS = 100000
N_FEAT = 128
N_SEG = 64
BLK = 80
N_BLOCKS = N_ROWS // BLK
SUB = 5
SB_ROWS = BLK * SUB
N_CORES = 2
N_SUBCORES = 16
N_WORKERS = N_CORES * N_SUBCORES
ROWS_SC = 51200
N_SB = ROWS_SC // SB_ROWS
SB_PER_W = N_SB // N_WORKERS
ROWS_TC = N_ROWS - ROWS_SC
B_TC = 800
NB_TC = ROWS_TC // B_TC
LANES = 16


def _sc_segment_sum(x, batch32):
    mesh = plsc.VectorSubcoreMesh(core_axis_name="c", subcore_axis_name="s")

    @functools.partial(
        pl.kernel,
        mesh=mesh,
        out_type=[
            jax.ShapeDtypeStruct((N_CORES, N_SEG, N_FEAT), jnp.float32),
            jax.ShapeDtypeStruct((N_CORES, N_SEG), jnp.float32),
        ],
        scratch_types=[
            pltpu.VMEM((SB_ROWS, N_FEAT), jnp.float32),
            pltpu.VMEM((SB_ROWS, N_FEAT), jnp.float32),
        ] + [pltpu.VMEM((BLK,), jnp.int32)] * (2 * SUB) + [
            pltpu.VMEM((BLK,), jnp.float32),
            pltpu.VMEM((N_SEG, N_FEAT), jnp.float32),
            pltpu.VMEM((N_SEG,), jnp.float32),
            pltpu.VMEM_SHARED((N_SEG, N_FEAT), jnp.float32),
            pltpu.VMEM_SHARED((N_SEG,), jnp.float32),
            pltpu.SemaphoreType.DMA,
            pltpu.SemaphoreType.DMA,
            pltpu.SemaphoreType.DMA,
            pltpu.SemaphoreType.DMA,
        ],
    )
    def seg_sum(x_hbm, b_hbm, sums_hbm, cnts_hbm,
                xb0, xb1, *rest):
        ib0 = rest[0:SUB]
        ib1 = rest[SUB:2 * SUB]
        (ones, zrow, zcnt, acc_sh, cnt_sh,
         sl0, sl1, ss0, ss1) = rest[2 * SUB:]
        cid = lax.axis_index("c")
        sid = lax.axis_index("s")
        wid = sid * N_CORES + cid

        one16 = jnp.full((LANES,), 1.0, dtype=jnp.float32)
        for k in range(BLK // LANES):
            ones[pl.ds(k * LANES, LANES)] = one16

        @pl.when(sid == 0)
        def _init():
            z16 = jnp.zeros((LANES,), dtype=jnp.float32)
            for k in range(N_SEG // LANES):
                zcnt[pl.ds(k * LANES, LANES)] = z16

            def zero_row(r, carry):
                for j in range(N_FEAT // LANES):
                    zrow[r, pl.ds(j * LANES, LANES)] = z16
                return carry

            lax.fori_loop(0, N_SEG, zero_row, 0)
            pltpu.sync_copy(zrow, acc_sh)
            pltpu.sync_copy(zcnt, cnt_sh)

        plsc.subcore_barrier()

        def srcs(i):
            sb = wid + i * N_WORKERS
            off = sb * SB_ROWS
            return (x_hbm.at[pl.ds(off, SB_ROWS)],
                    [b_hbm.at[pl.ds(off + j * BLK, BLK)] for j in range(SUB)])

        def load_start(i, xb, ib, sl):
            xs, bs = srcs(i)
            pltpu.async_copy(xs, xb, sl)
            for j in range(SUB):
                pltpu.async_copy(bs[j], ib[j], sl)

        scatter_descs = {}

        def load_wait_and_scatter(i, xb, ib, sl, ss):
            xs, bs = srcs(i)
            pltpu.make_async_copy(xs, xb, sl).wait()
            for j in range(SUB):
                pltpu.make_async_copy(bs[j], ib[j], sl).wait()
            ds = []
            for j in range(SUB):
                ds.append(pltpu.async_copy(
                    xb.at[pl.ds(j * BLK, BLK)],
                    acc_sh.at[ib[j]], ss, add=True))
                ds.append(pltpu.async_copy(ones, cnt_sh.at[ib[j]],
                                           ss, add=True))
            scatter_descs[i] = ds

        def scatter_drain(i):
            if i < 0 or i not in scatter_descs:
                return
            for d in scatter_descs[i]:
                d.wait()

        bufs = [(xb0, ib0, sl0, ss0), (xb1, ib1, sl1, ss1)]
        load_start(0, *bufs[0][:3])
        for i in range(SB_PER_W):
            if i + 1 < SB_PER_W:
                scatter_drain(i - 1)
                load_start(i + 1, *bufs[(i + 1) % 2][:3])
            load_wait_and_scatter(i, *bufs[i % 2])
        scatter_drain(SB_PER_W - 2)
        scatter_drain(SB_PER_W - 1)

        plsc.subcore_barrier()

        @pl.when(sid == 0)
        def _emit():
            pltpu.sync_copy(acc_sh, sums_hbm.at[cid])
            pltpu.sync_copy(cnt_sh, cnts_hbm.at[cid])

    return seg_sum(x, batch32)


def _tc_tail_kernel(ids_ref, x_ref, sums_ref, cnts_ref):
    @pl.when(pl.program_id(0) == 0)
    def _init():
        sums_ref[...] = jnp.zeros_like(sums_ref)
        cnts_ref[...] = jnp.zeros_like(cnts_ref)

    ids = ids_ref[0, 0, :]
    onehot = (lax.broadcasted_iota(jnp.int32, (N_SEG, B_TC), 0)
              == ids[None, :]).astype(jnp.float32)
    sums_ref[...] += jnp.dot(onehot, x_ref[...],
                             preferred_element_type=jnp.float32)
    cnts_ref[...] += jnp.sum(onehot, axis=1, keepdims=False)[None, :]


def _tc_tail_segment_sum(x, ids3d):
    return pl.pallas_call(
        _tc_tail_kernel,
        grid=(NB_TC,),
        in_specs=[
            pl.BlockSpec((1, 1, B_TC), lambda i: (i, 0, 0)),
            pl.BlockSpec((B_TC, N_FEAT), lambda i: (i + ROWS_SC // B_TC, 0)),
        ],
        out_specs=[
            pl.BlockSpec((N_SEG, N_FEAT), lambda i: (0, 0)),
            pl.BlockSpec((1, N_SEG), lambda i: (0, 0)),
        ],
        out_shape=[
            jax.ShapeDtypeStruct((N_SEG, N_FEAT), jnp.float32),
            jax.ShapeDtypeStruct((1, N_SEG), jnp.float32),
        ],
    )(ids3d, x)


def _combine_kernel(sums_ref, cnts_ref, tsums_ref, tcnts_ref, out_ref):
    s = sums_ref[0] + sums_ref[1] + tsums_ref[...]
    c = jnp.maximum(cnts_ref[0] + cnts_ref[1] + tcnts_ref[0], 1.0)
    out_ref[...] = s / c[:, None]


def _tc_combine(sums, cnts, tsums, tcnts):
    return pl.pallas_call(
        _combine_kernel,
        out_shape=jax.ShapeDtypeStruct((N_SEG, N_FEAT), jnp.float32),
    )(sums, cnts, tsums, tcnts)


@jax.jit
def kernel(x, batch):
    batch32 = batch.astype(jnp.int32)
    ids3d = batch32[ROWS_SC:].reshape(NB_TC, 1, B_TC)
    sums, cnts = _sc_segment_sum(x, batch32)
    tsums, tcnts = _tc_tail_segment_sum(x, ids3d)
    return _tc_combine(sums, cnts, tsums, tcnts)


# speedup vs baseline: 2.0766x; 1.0259x over previous
import functools

import jax
import jax.numpy as jnp
from jax import lax
from jax.experimental import pallas as pl
from jax.experimental.pallas import tpu as pltpu
from jax.experimental.pallas import tpu_sc as plsc

N_ROWS = 100000
N_FEAT = 128
N_SEG = 64
BLK = 80
N_BLOCKS = N_ROWS // BLK
SUB = 5
SB_ROWS = BLK * SUB
N_SB = N_ROWS // SB_ROWS
N_CORES = 2
N_SUBCORES = 16
N_WORKERS = N_CORES * N_SUBCORES
SB_PER_W = -(-N_SB // N_WORKERS)
LANES = 16


def _sc_segment_sum(x, batch32):
    mesh = plsc.VectorSubcoreMesh(core_axis_name="c", subcore_axis_name="s")

    @functools.partial(
        pl.kernel,
        mesh=mesh,
        out_type=[
            jax.ShapeDtypeStruct((N_CORES, N_SEG, N_FEAT), jnp.float32),
            jax.ShapeDtypeStruct((N_CORES, N_SEG), jnp.float32),
        ],
        scratch_types=[
            pltpu.VMEM((SB_ROWS, N_FEAT), jnp.float32),
            pltpu.VMEM((SB_ROWS, N_FEAT), jnp.float32),
        ] + [pltpu.VMEM((BLK,), jnp.int32)] * (2 * SUB) + [
            pltpu.VMEM((BLK,), jnp.float32),
            pltpu.VMEM((N_SEG, N_FEAT), jnp.float32),
            pltpu.VMEM((N_SEG,), jnp.float32),
            pltpu.VMEM_SHARED((N_SEG, N_FEAT), jnp.float32),
            pltpu.VMEM_SHARED((N_SEG,), jnp.float32),
            pltpu.SemaphoreType.DMA,
            pltpu.SemaphoreType.DMA,
            pltpu.SemaphoreType.DMA,
            pltpu.SemaphoreType.DMA,
        ],
    )
    def seg_sum(x_hbm, b_hbm, sums_hbm, cnts_hbm,
                xb0, xb1, *rest):
        ib0 = rest[0:SUB]
        ib1 = rest[SUB:2 * SUB]
        (ones, zrow, zcnt, acc_sh, cnt_sh,
         sl0, sl1, ss0, ss1) = rest[2 * SUB:]
        cid = lax.axis_index("c")
        sid = lax.axis_index("s")
        wid = sid * N_CORES + cid

        one16 = jnp.full((LANES,), 1.0, dtype=jnp.float32)
        for k in range(BLK // LANES):
            ones[pl.ds(k * LANES, LANES)] = one16

        @pl.when(sid == 0)
        def _init():
            z16 = jnp.zeros((LANES,), dtype=jnp.float32)
            for k in range(N_SEG // LANES):
                zcnt[pl.ds(k * LANES, LANES)] = z16

            def zero_row(r, carry):
                for j in range(N_FEAT // LANES):
                    zrow[r, pl.ds(j * LANES, LANES)] = z16
                return carry

            lax.fori_loop(0, N_SEG, zero_row, 0)
            pltpu.sync_copy(zrow, acc_sh)
            pltpu.sync_copy(zcnt, cnt_sh)

        plsc.subcore_barrier()

        def srcs(i):
            sb = wid + i * N_WORKERS
            off = sb * SB_ROWS
            return (x_hbm.at[pl.ds(off, SB_ROWS)],
                    [b_hbm.at[pl.ds(off + j * BLK, BLK)] for j in range(SUB)])

        HALF = SB_ROWS // 2

        def load_start(i, xb, ib, sl):
            sb = wid + i * N_WORKERS

            @pl.when(sb < N_SB)
            def _():
                off = sb * SB_ROWS
                pltpu.async_copy(x_hbm.at[pl.ds(off, HALF)],
                                 xb.at[pl.ds(0, HALF)], sl)
                pltpu.async_copy(x_hbm.at[pl.ds(off + HALF, HALF)],
                                 xb.at[pl.ds(HALF, HALF)], sl)
                _, bs = srcs(i)
                for j in range(SUB):
                    pltpu.async_copy(bs[j], ib[j], sl)

        scatter_descs = {}

        def load_wait_and_scatter(i, xb, ib, sl, ss):
            sb = wid + i * N_WORKERS

            @pl.when(sb < N_SB)
            def _():
                off = (wid + i * N_WORKERS) * SB_ROWS
                pltpu.make_async_copy(x_hbm.at[pl.ds(off, HALF)],
                                      xb.at[pl.ds(0, HALF)], sl).wait()
                pltpu.make_async_copy(x_hbm.at[pl.ds(off + HALF, HALF)],
                                      xb.at[pl.ds(HALF, HALF)], sl).wait()
                _, bs = srcs(i)
                for j in range(SUB):
                    pltpu.make_async_copy(bs[j], ib[j], sl).wait()
                ds = []
                for j in range(SUB):
                    ds.append(pltpu.async_copy(
                        xb.at[pl.ds(j * BLK, BLK)],
                        acc_sh.at[ib[j]], ss, add=True))
                    ds.append(pltpu.async_copy(ones, cnt_sh.at[ib[j]],
                                               ss, add=True))
                scatter_descs[i] = ds

        def scatter_drain(i):
            if i < 0 or i not in scatter_descs:
                return
            sb = wid + i * N_WORKERS

            @pl.when(sb < N_SB)
            def _():
                for d in scatter_descs[i]:
                    d.wait()

        bufs = [(xb0, ib0, sl0, ss0), (xb1, ib1, sl1, ss1)]
        load_start(0, *bufs[0][:3])
        for i in range(SB_PER_W):
            if i + 1 < SB_PER_W:
                scatter_drain(i - 1)
                load_start(i + 1, *bufs[(i + 1) % 2][:3])
            load_wait_and_scatter(i, *bufs[i % 2])
        scatter_drain(SB_PER_W - 2)
        scatter_drain(SB_PER_W - 1)

        plsc.subcore_barrier()

        @pl.when(sid == 0)
        def _emit():
            pltpu.sync_copy(acc_sh, sums_hbm.at[cid])
            pltpu.sync_copy(cnt_sh, cnts_hbm.at[cid])

    return seg_sum(x, batch32)


def _combine_kernel(sums_ref, cnts_ref, out_ref):
    s = sums_ref[0] + sums_ref[1]
    c = jnp.maximum(cnts_ref[0] + cnts_ref[1], 1.0)
    out_ref[...] = s / c[:, None]


def _tc_combine(sums, cnts):
    return pl.pallas_call(
        _combine_kernel,
        out_shape=jax.ShapeDtypeStruct((N_SEG, N_FEAT), jnp.float32),
    )(sums, cnts)


@jax.jit
def kernel(x, batch):
    batch32 = batch.astype(jnp.int32)
    sums, cnts = _sc_segment_sum(x, batch32)
    return _tc_combine(sums, cnts)


# speedup vs baseline: 2.3361x; 1.1250x over previous
import functools

import jax
import jax.numpy as jnp
from jax import lax
from jax.experimental import pallas as pl
from jax.experimental.pallas import tpu as pltpu
from jax.experimental.pallas import tpu_sc as plsc

N_ROWS = 100000
N_FEAT = 128
N_SEG = 64
BLK = 80
SUB = 5
SB_ROWS = BLK * SUB
N_CORES = 2
N_SUBCORES = 16
N_WORKERS = N_CORES * N_SUBCORES
ROWS_SC = 61600
N_SB = ROWS_SC // SB_ROWS
SB_PER_W = -(-N_SB // N_WORKERS)
ROWS_TC = N_ROWS - ROWS_SC
B_TC = 800
NB_TC = ROWS_TC // B_TC
LANES = 16


def _sc_segment_sum(x, batch32):
    mesh = plsc.VectorSubcoreMesh(core_axis_name="c", subcore_axis_name="s")

    @functools.partial(
        pl.kernel,
        mesh=mesh,
        out_type=[
            jax.ShapeDtypeStruct((N_CORES, N_SEG, N_FEAT), jnp.float32),
            jax.ShapeDtypeStruct((N_CORES, N_SEG), jnp.float32),
        ],
        scratch_types=[
            pltpu.VMEM((SB_ROWS, N_FEAT), jnp.float32),
            pltpu.VMEM((SB_ROWS, N_FEAT), jnp.float32),
        ] + [pltpu.VMEM((BLK,), jnp.int32)] * (2 * SUB) + [
            pltpu.VMEM((BLK,), jnp.float32),
            pltpu.VMEM((N_SEG, N_FEAT), jnp.float32),
            pltpu.VMEM((N_SEG,), jnp.float32),
            pltpu.VMEM_SHARED((N_SEG, N_FEAT), jnp.float32),
            pltpu.VMEM_SHARED((N_SEG,), jnp.float32),
            pltpu.SemaphoreType.DMA,
            pltpu.SemaphoreType.DMA,
            pltpu.SemaphoreType.DMA,
            pltpu.SemaphoreType.DMA,
        ],
    )
    def seg_sum(x_hbm, b_hbm, sums_hbm, cnts_hbm,
                xb0, xb1, *rest):
        ib0 = rest[0:SUB]
        ib1 = rest[SUB:2 * SUB]
        (ones, zrow, zcnt, acc_sh, cnt_sh,
         sl0, sl1, ss0, ss1) = rest[2 * SUB:]
        cid = lax.axis_index("c")
        sid = lax.axis_index("s")
        wid = sid * N_CORES + cid

        one16 = jnp.full((LANES,), 1.0, dtype=jnp.float32)
        for k in range(BLK // LANES):
            ones[pl.ds(k * LANES, LANES)] = one16

        @pl.when(sid == 0)
        def _init():
            z16 = jnp.zeros((LANES,), dtype=jnp.float32)
            for k in range(N_SEG // LANES):
                zcnt[pl.ds(k * LANES, LANES)] = z16

            def zero_row(r, carry):
                for j in range(N_FEAT // LANES):
                    zrow[r, pl.ds(j * LANES, LANES)] = z16
                return carry

            lax.fori_loop(0, N_SEG, zero_row, 0)
            pltpu.sync_copy(zrow, acc_sh)
            pltpu.sync_copy(zcnt, cnt_sh)

        plsc.subcore_barrier()

        def srcs(i):
            sb = wid + i * N_WORKERS
            off = sb * SB_ROWS
            return (x_hbm.at[pl.ds(off, SB_ROWS)],
                    [b_hbm.at[pl.ds(off + j * BLK, BLK)] for j in range(SUB)])

        def load_start(i, xb, ib, sl):
            sb = wid + i * N_WORKERS

            @pl.when(sb < N_SB)
            def _():
                xs, bs = srcs(i)
                pltpu.async_copy(xs, xb, sl)
                for j in range(SUB):
                    pltpu.async_copy(bs[j], ib[j], sl)

        scatter_descs = {}

        def load_wait_and_scatter(i, xb, ib, sl, ss):
            sb = wid + i * N_WORKERS

            @pl.when(sb < N_SB)
            def _():
                xs, bs = srcs(i)
                pltpu.make_async_copy(xs, xb, sl).wait()
                for j in range(SUB):
                    pltpu.make_async_copy(bs[j], ib[j], sl).wait()
                ds = []
                for j in range(SUB):
                    ds.append(pltpu.async_copy(
                        xb.at[pl.ds(j * BLK, BLK)],
                        acc_sh.at[ib[j]], ss, add=True))
                    ds.append(pltpu.async_copy(ones, cnt_sh.at[ib[j]],
                                               ss, add=True))
                scatter_descs[i] = ds

        def scatter_drain(i):
            if i < 0 or i not in scatter_descs:
                return
            sb = wid + i * N_WORKERS

            @pl.when(sb < N_SB)
            def _():
                for d in scatter_descs[i]:
                    d.wait()

        bufs = [(xb0, ib0, sl0, ss0), (xb1, ib1, sl1, ss1)]
        load_start(0, *bufs[0][:3])
        for i in range(SB_PER_W):
            if i + 1 < SB_PER_W:
                scatter_drain(i - 1)
                load_start(i + 1, *bufs[(i + 1) % 2][:3])
            load_wait_and_scatter(i, *bufs[i % 2])
        scatter_drain(SB_PER_W - 2)
        scatter_drain(SB_PER_W - 1)

        plsc.subcore_barrier()

        @pl.when(sid == 0)
        def _emit():
            pltpu.sync_copy(acc_sh, sums_hbm.at[cid])
            pltpu.sync_copy(cnt_sh, cnts_hbm.at[cid])

    return seg_sum(x, batch32)


def _tc_tail_kernel(ids_ref, x_ref, sums_ref, cnts_ref):
    @pl.when(pl.program_id(0) == 0)
    def _init():
        sums_ref[...] = jnp.zeros_like(sums_ref)
        cnts_ref[...] = jnp.zeros_like(cnts_ref)

    ids = ids_ref[0, 0, :]
    onehot = (lax.broadcasted_iota(jnp.int32, (N_SEG, B_TC), 0)
              == ids[None, :]).astype(jnp.float32)
    sums_ref[...] += jnp.dot(onehot, x_ref[...],
                             preferred_element_type=jnp.float32)
    cnts_ref[...] += jnp.sum(onehot, axis=1, keepdims=False)[None, :]


def _tc_tail_segment_sum(x, ids3d):
    return pl.pallas_call(
        _tc_tail_kernel,
        grid=(NB_TC,),
        in_specs=[
            pl.BlockSpec((1, 1, B_TC), lambda i: (i, 0, 0)),
            pl.BlockSpec((B_TC, N_FEAT), lambda i: (i + ROWS_SC // B_TC, 0)),
        ],
        out_specs=[
            pl.BlockSpec((N_SEG, N_FEAT), lambda i: (0, 0)),
            pl.BlockSpec((1, N_SEG), lambda i: (0, 0)),
        ],
        out_shape=[
            jax.ShapeDtypeStruct((N_SEG, N_FEAT), jnp.float32),
            jax.ShapeDtypeStruct((1, N_SEG), jnp.float32),
        ],
    )(ids3d, x)


def _combine_kernel(sums_ref, cnts_ref, tsums_ref, tcnts_ref, out_ref):
    s = sums_ref[0] + sums_ref[1] + tsums_ref[...]
    c = jnp.maximum(cnts_ref[0] + cnts_ref[1] + tcnts_ref[0], 1.0)
    out_ref[...] = s / c[:, None]


def _tc_combine(sums, cnts, tsums, tcnts):
    return pl.pallas_call(
        _combine_kernel,
        out_shape=jax.ShapeDtypeStruct((N_SEG, N_FEAT), jnp.float32),
    )(sums, cnts, tsums, tcnts)


@jax.jit
def kernel(x, batch):
    batch32 = batch.astype(jnp.int32)
    ids3d = batch32[ROWS_SC:].reshape(NB_TC, 1, B_TC)
    sums, cnts = _sc_segment_sum(x, batch32)
    tsums, tcnts = _tc_tail_segment_sum(x, ids3d)
    return _tc_combine(sums, cnts, tsums, tcnts)


# speedup vs baseline: 2.5634x; 1.0973x over previous
import functools

import jax
import jax.numpy as jnp
from jax import lax
from jax.experimental import pallas as pl
from jax.experimental.pallas import tpu as pltpu
from jax.experimental.pallas import tpu_sc as plsc

N_ROWS = 100000
N_FEAT = 128
N_SEG = 64
BLK = 80
SUB = 5
SB_ROWS = BLK * SUB
N_CORES = 2
N_SUBCORES = 16
N_WORKERS = N_CORES * N_SUBCORES
ROWS_SC = 62000
N_SB = ROWS_SC // SB_ROWS
SB_PER_W = -(-N_SB // N_WORKERS)
ROWS_TC = N_ROWS - ROWS_SC
B_TC = 2000
NB_TC = ROWS_TC // B_TC
LANES = 16


def _sc_segment_sum(x, batch32):
    mesh = plsc.VectorSubcoreMesh(core_axis_name="c", subcore_axis_name="s")

    @functools.partial(
        pl.kernel,
        mesh=mesh,
        out_type=[
            jax.ShapeDtypeStruct((N_CORES, N_SEG, N_FEAT), jnp.float32),
            jax.ShapeDtypeStruct((N_CORES, N_SEG), jnp.float32),
        ],
        scratch_types=[
            pltpu.VMEM((SB_ROWS, N_FEAT), jnp.float32),
            pltpu.VMEM((SB_ROWS, N_FEAT), jnp.float32),
        ] + [pltpu.VMEM((BLK,), jnp.int32)] * (2 * SUB) + [
            pltpu.VMEM((BLK,), jnp.float32),
            pltpu.VMEM((N_SEG, N_FEAT), jnp.float32),
            pltpu.VMEM((N_SEG,), jnp.float32),
            pltpu.VMEM_SHARED((N_SEG, N_FEAT), jnp.float32),
            pltpu.VMEM_SHARED((N_SEG,), jnp.float32),
            pltpu.SemaphoreType.DMA,
            pltpu.SemaphoreType.DMA,
            pltpu.SemaphoreType.DMA,
            pltpu.SemaphoreType.DMA,
        ],
    )
    def seg_sum(x_hbm, b_hbm, sums_hbm, cnts_hbm,
                xb0, xb1, *rest):
        ib0 = rest[0:SUB]
        ib1 = rest[SUB:2 * SUB]
        (ones, zrow, zcnt, acc_sh, cnt_sh,
         sl0, sl1, ss0, ss1) = rest[2 * SUB:]
        cid = lax.axis_index("c")
        sid = lax.axis_index("s")
        wid = sid * N_CORES + cid

        one16 = jnp.full((LANES,), 1.0, dtype=jnp.float32)
        for k in range(BLK // LANES):
            ones[pl.ds(k * LANES, LANES)] = one16

        @pl.when(sid == 0)
        def _init():
            z16 = jnp.zeros((LANES,), dtype=jnp.float32)
            for k in range(N_SEG // LANES):
                zcnt[pl.ds(k * LANES, LANES)] = z16

            def zero_row(r, carry):
                for j in range(N_FEAT // LANES):
                    zrow[r, pl.ds(j * LANES, LANES)] = z16
                return carry

            lax.fori_loop(0, N_SEG, zero_row, 0)
            pltpu.sync_copy(zrow, acc_sh)
            pltpu.sync_copy(zcnt, cnt_sh)

        plsc.subcore_barrier()

        def srcs(i):
            sb = wid + i * N_WORKERS
            off = sb * SB_ROWS
            return (x_hbm.at[pl.ds(off, SB_ROWS)],
                    [b_hbm.at[pl.ds(off + j * BLK, BLK)] for j in range(SUB)])

        def load_start(i, xb, ib, sl):
            sb = wid + i * N_WORKERS

            @pl.when(sb < N_SB)
            def _():
                xs, bs = srcs(i)
                pltpu.async_copy(xs, xb, sl)
                for j in range(SUB):
                    pltpu.async_copy(bs[j], ib[j], sl)

        scatter_descs = {}

        def load_wait_and_scatter(i, xb, ib, sl, ss):
            sb = wid + i * N_WORKERS

            @pl.when(sb < N_SB)
            def _():
                xs, bs = srcs(i)
                pltpu.make_async_copy(xs, xb, sl).wait()
                for j in range(SUB):
                    pltpu.make_async_copy(bs[j], ib[j], sl).wait()
                ds = []
                for j in range(SUB):
                    ds.append(pltpu.async_copy(
                        xb.at[pl.ds(j * BLK, BLK)],
                        acc_sh.at[ib[j]], ss, add=True))
                    ds.append(pltpu.async_copy(ones, cnt_sh.at[ib[j]],
                                               ss, add=True))
                scatter_descs[i] = ds

        def scatter_drain(i):
            if i < 0 or i not in scatter_descs:
                return
            sb = wid + i * N_WORKERS

            @pl.when(sb < N_SB)
            def _():
                for d in scatter_descs[i]:
                    d.wait()

        bufs = [(xb0, ib0, sl0, ss0), (xb1, ib1, sl1, ss1)]
        load_start(0, *bufs[0][:3])
        for i in range(SB_PER_W):
            if i + 1 < SB_PER_W:
                scatter_drain(i - 1)
                load_start(i + 1, *bufs[(i + 1) % 2][:3])
            load_wait_and_scatter(i, *bufs[i % 2])
        scatter_drain(SB_PER_W - 2)
        scatter_drain(SB_PER_W - 1)

        plsc.subcore_barrier()

        @pl.when(sid == 0)
        def _emit():
            pltpu.sync_copy(acc_sh, sums_hbm.at[cid])
            pltpu.sync_copy(cnt_sh, cnts_hbm.at[cid])

    return seg_sum(x, batch32)


def _tc_tail_kernel(ids_ref, x_ref, sums_ref, cnts_ref):
    @pl.when(pl.program_id(0) == 0)
    def _init():
        sums_ref[...] = jnp.zeros_like(sums_ref)
        cnts_ref[...] = jnp.zeros_like(cnts_ref)

    ids = ids_ref[0, 0, :]
    onehot = (lax.broadcasted_iota(jnp.int32, (N_SEG, B_TC), 0)
              == ids[None, :]).astype(jnp.float32)
    sums_ref[...] += jnp.dot(onehot, x_ref[...],
                             preferred_element_type=jnp.float32)
    cnts_ref[...] += jnp.sum(onehot, axis=1, keepdims=False)[None, :]


def _tc_tail_segment_sum(x, ids3d):
    return pl.pallas_call(
        _tc_tail_kernel,
        grid=(NB_TC,),
        in_specs=[
            pl.BlockSpec((1, 1, B_TC), lambda i: (i, 0, 0)),
            pl.BlockSpec((B_TC, N_FEAT), lambda i: (i + ROWS_SC // B_TC, 0)),
        ],
        out_specs=[
            pl.BlockSpec((N_SEG, N_FEAT), lambda i: (0, 0)),
            pl.BlockSpec((1, N_SEG), lambda i: (0, 0)),
        ],
        out_shape=[
            jax.ShapeDtypeStruct((N_SEG, N_FEAT), jnp.float32),
            jax.ShapeDtypeStruct((1, N_SEG), jnp.float32),
        ],
    )(ids3d, x)


def _combine_kernel(sums_ref, cnts_ref, tsums_ref, tcnts_ref, out_ref):
    s = sums_ref[0] + sums_ref[1] + tsums_ref[...]
    c = jnp.maximum(cnts_ref[0] + cnts_ref[1] + tcnts_ref[0], 1.0)
    out_ref[...] = s / c[:, None]


def _tc_combine(sums, cnts, tsums, tcnts):
    return pl.pallas_call(
        _combine_kernel,
        out_shape=jax.ShapeDtypeStruct((N_SEG, N_FEAT), jnp.float32),
    )(sums, cnts, tsums, tcnts)


@jax.jit
def kernel(x, batch):
    batch32 = batch.astype(jnp.int32)
    ids3d = batch32[ROWS_SC:].reshape(NB_TC, 1, B_TC)
    sums, cnts = _sc_segment_sum(x, batch32)
    tsums, tcnts = _tc_tail_segment_sum(x, ids3d)
    return _tc_combine(sums, cnts, tsums, tcnts)


# speedup vs baseline: 2.5819x; 1.0072x over previous
import functools

import jax
import jax.numpy as jnp
from jax import lax
from jax.experimental import pallas as pl
from jax.experimental.pallas import tpu as pltpu
from jax.experimental.pallas import tpu_sc as plsc

N_ROWS = 100000
N_FEAT = 128
N_SEG = 64
BLK = 80
SUB = 5
SB_ROWS = BLK * SUB
N_CORES = 2
N_SUBCORES = 16
N_WORKERS = N_CORES * N_SUBCORES
ROWS_SC = 56000
N_SB = ROWS_SC // SB_ROWS
SB_PER_W = -(-N_SB // N_WORKERS)
ROWS_TC = N_ROWS - ROWS_SC
B_TC = 2000
NB_TC = ROWS_TC // B_TC
LANES = 16


def _sc_segment_sum(x, batch32):
    mesh = plsc.VectorSubcoreMesh(core_axis_name="c", subcore_axis_name="s")

    @functools.partial(
        pl.kernel,
        mesh=mesh,
        out_type=[
            jax.ShapeDtypeStruct((N_CORES, N_SEG, N_FEAT), jnp.float32),
            jax.ShapeDtypeStruct((N_CORES, N_SEG), jnp.float32),
        ],
        scratch_types=[
            pltpu.VMEM((SB_ROWS, N_FEAT), jnp.float32),
            pltpu.VMEM((SB_ROWS, N_FEAT), jnp.float32),
        ] + [pltpu.VMEM((BLK,), jnp.int32)] * (2 * SUB) + [
            pltpu.VMEM((BLK,), jnp.float32),
            pltpu.VMEM((N_SEG, N_FEAT), jnp.float32),
            pltpu.VMEM((N_SEG,), jnp.float32),
            pltpu.VMEM_SHARED((N_SEG, N_FEAT), jnp.float32),
            pltpu.VMEM_SHARED((N_SEG,), jnp.float32),
            pltpu.SemaphoreType.DMA,
            pltpu.SemaphoreType.DMA,
            pltpu.SemaphoreType.DMA,
            pltpu.SemaphoreType.DMA,
        ],
    )
    def seg_sum(x_hbm, b_hbm, sums_hbm, cnts_hbm,
                xb0, xb1, *rest):
        ib0 = rest[0:SUB]
        ib1 = rest[SUB:2 * SUB]
        (ones, zrow, zcnt, acc_sh, cnt_sh,
         sl0, sl1, ss0, ss1) = rest[2 * SUB:]
        cid = lax.axis_index("c")
        sid = lax.axis_index("s")
        wid = sid * N_CORES + cid

        one16 = jnp.full((LANES,), 1.0, dtype=jnp.float32)
        for k in range(BLK // LANES):
            ones[pl.ds(k * LANES, LANES)] = one16

        @pl.when(sid == 0)
        def _init():
            z16 = jnp.zeros((LANES,), dtype=jnp.float32)
            for k in range(N_SEG // LANES):
                zcnt[pl.ds(k * LANES, LANES)] = z16

            def zero_row(r, carry):
                for j in range(N_FEAT // LANES):
                    zrow[r, pl.ds(j * LANES, LANES)] = z16
                return carry

            lax.fori_loop(0, N_SEG, zero_row, 0)
            pltpu.sync_copy(zrow, acc_sh)
            pltpu.sync_copy(zcnt, cnt_sh)

        plsc.subcore_barrier()

        def srcs(i):
            sb = wid + i * N_WORKERS
            off = sb * SB_ROWS
            return (x_hbm.at[pl.ds(off, SB_ROWS)],
                    [b_hbm.at[pl.ds(off + j * BLK, BLK)] for j in range(SUB)])

        def load_start(i, xb, ib, sl):
            sb = wid + i * N_WORKERS

            @pl.when(sb < N_SB)
            def _():
                xs, bs = srcs(i)
                pltpu.async_copy(xs, xb, sl)
                for j in range(SUB):
                    pltpu.async_copy(bs[j], ib[j], sl)

        scatter_descs = {}

        def load_wait_and_scatter(i, xb, ib, sl, ss):
            sb = wid + i * N_WORKERS

            @pl.when(sb < N_SB)
            def _():
                xs, bs = srcs(i)
                pltpu.make_async_copy(xs, xb, sl).wait()
                for j in range(SUB):
                    pltpu.make_async_copy(bs[j], ib[j], sl).wait()
                ds = []
                for j in range(SUB):
                    ds.append(pltpu.async_copy(
                        xb.at[pl.ds(j * BLK, BLK)],
                        acc_sh.at[ib[j]], ss, add=True))
                    ds.append(pltpu.async_copy(ones, cnt_sh.at[ib[j]],
                                               ss, add=True))
                scatter_descs[i] = ds

        def scatter_drain(i):
            if i < 0 or i not in scatter_descs:
                return
            sb = wid + i * N_WORKERS

            @pl.when(sb < N_SB)
            def _():
                for d in scatter_descs[i]:
                    d.wait()

        bufs = [(xb0, ib0, sl0, ss0), (xb1, ib1, sl1, ss1)]
        load_start(0, *bufs[0][:3])
        for i in range(SB_PER_W):
            if i + 1 < SB_PER_W:
                scatter_drain(i - 1)
                load_start(i + 1, *bufs[(i + 1) % 2][:3])
            load_wait_and_scatter(i, *bufs[i % 2])
        scatter_drain(SB_PER_W - 2)
        scatter_drain(SB_PER_W - 1)

        plsc.subcore_barrier()

        @pl.when(sid == 0)
        def _emit():
            pltpu.sync_copy(acc_sh, sums_hbm.at[cid])
            pltpu.sync_copy(cnt_sh, cnts_hbm.at[cid])

    return seg_sum(x, batch32)


def _tc_tail_kernel(ids_ref, x_ref, sums_ref, cnts_ref):
    @pl.when(pl.program_id(0) == 0)
    def _init():
        sums_ref[...] = jnp.zeros_like(sums_ref)
        cnts_ref[...] = jnp.zeros_like(cnts_ref)

    ids = ids_ref[0, 0, :]
    onehot = (lax.broadcasted_iota(jnp.int32, (N_SEG, B_TC), 0)
              == ids[None, :]).astype(jnp.float32)
    sums_ref[...] += jnp.dot(onehot, x_ref[...],
                             preferred_element_type=jnp.float32)
    cnts_ref[...] += jnp.sum(onehot, axis=1, keepdims=False)[None, :]


def _tc_tail_segment_sum(x, ids3d):
    return pl.pallas_call(
        _tc_tail_kernel,
        grid=(NB_TC,),
        in_specs=[
            pl.BlockSpec((1, 1, B_TC), lambda i: (i, 0, 0)),
            pl.BlockSpec((B_TC, N_FEAT), lambda i: (i + ROWS_SC // B_TC, 0)),
        ],
        out_specs=[
            pl.BlockSpec((N_SEG, N_FEAT), lambda i: (0, 0)),
            pl.BlockSpec((1, N_SEG), lambda i: (0, 0)),
        ],
        out_shape=[
            jax.ShapeDtypeStruct((N_SEG, N_FEAT), jnp.float32),
            jax.ShapeDtypeStruct((1, N_SEG), jnp.float32),
        ],
    )(ids3d, x)


def _combine_kernel(sums_ref, cnts_ref, tsums_ref, tcnts_ref, out_ref):
    s = sums_ref[0] + sums_ref[1] + tsums_ref[...]
    c = jnp.maximum(cnts_ref[0] + cnts_ref[1] + tcnts_ref[0], 1.0)
    out_ref[...] = s / c[:, None]


def _tc_combine(sums, cnts, tsums, tcnts):
    return pl.pallas_call(
        _combine_kernel,
        out_shape=jax.ShapeDtypeStruct((N_SEG, N_FEAT), jnp.float32),
    )(sums, cnts, tsums, tcnts)


@jax.jit
def kernel(x, batch):
    batch32 = batch.astype(jnp.int32)
    ids3d = batch32[ROWS_SC:].reshape(NB_TC, 1, B_TC)
    sums, cnts = _sc_segment_sum(x, batch32)
    tsums, tcnts = _tc_tail_segment_sum(x, ids3d)
    return _tc_combine(sums, cnts, tsums, tcnts)
